# baseline (speedup 1.0000x reference)
import jax
import jax.numpy as jnp
from jax.experimental import pallas as pl
from jax.experimental.pallas import tpu as pltpu


def _dot(a, b):
    return jax.lax.dot_general(
        a, b, (((1,), (0,)), ((), ())),
        precision=jax.lax.Precision.DEFAULT,
        preferred_element_type=jnp.float32)


def _head_kernel(ea_ref, w1_ref, b1_ref, w2_ref, b2_ref, w3_ref, b3_ref,
                 w4_ref, b4_ref, out_ref):
    h = jnp.maximum(_dot(w1_ref[...], ea_ref[...]) + b1_ref[...], 0.0)
    h = jnp.maximum(_dot(w2_ref[...], h) + b2_ref[...], 0.0)
    h = jnp.maximum(_dot(w3_ref[...], h) + b3_ref[...], 0.0)
    o = jnp.sum(h * w4_ref[...], axis=0, keepdims=True)
    out_ref[...] = jax.nn.sigmoid(o + b4_ref[...])


def kernel(x, edge_index, edge_attr, params_rel, params_obj, params_w):
    E, DE = edge_attr.shape
    (W1, b1), (W2, b2), (W3, b3), (W4, b4) = params_w
    H1, H2, H3, DO = W1.shape[0], W2.shape[0], W3.shape[0], W4.shape[0]

    eaT = edge_attr.T

    lanes = 64000
    grid = (pl.cdiv(E, lanes),)

    out = pl.pallas_call(
        _head_kernel,
        grid=grid,
        in_specs=[
            pl.BlockSpec((DE, lanes), lambda i: (0, i)),
            pl.BlockSpec((H1, DE), lambda i: (0, 0)),
            pl.BlockSpec((H1, 1), lambda i: (0, 0)),
            pl.BlockSpec((H2, H1), lambda i: (0, 0)),
            pl.BlockSpec((H2, 1), lambda i: (0, 0)),
            pl.BlockSpec((H3, H2), lambda i: (0, 0)),
            pl.BlockSpec((H3, 1), lambda i: (0, 0)),
            pl.BlockSpec((H3, DO), lambda i: (0, 0)),
            pl.BlockSpec((DO, 1), lambda i: (0, 0)),
        ],
        out_specs=pl.BlockSpec((DO, lanes), lambda i: (0, i)),
        out_shape=jax.ShapeDtypeStruct((DO, E), jnp.float32),
        compiler_params=pltpu.CompilerParams(
            dimension_semantics=("parallel",)),
    )(eaT, W1, b1[:, None], W2, b2[:, None], W3, b3[:, None],
      W4.T, b4[:, None])
    return out.reshape(E, DO)


# speedup vs baseline: 1.0789x; 1.0789x over previous
import jax
import jax.numpy as jnp
from jax.experimental import pallas as pl
from jax.experimental.pallas import tpu as pltpu


def _dot(a, b):
    return jax.lax.dot_general(
        a, b, (((1,), (0,)), ((), ())),
        precision=jax.lax.Precision.DEFAULT,
        preferred_element_type=jnp.float32)


def _head_kernel(ea_ref, w1_ref, b1_ref, w2_ref, b2_ref, w3_ref, b3_ref,
                 w4_ref, b4_ref, out_ref):
    half = ea_ref.shape[1] // 2
    for s in (slice(0, half), slice(half, None)):
        h = jnp.maximum(_dot(w1_ref[...], ea_ref[:, s]) + b1_ref[...], 0.0)
        h = jnp.maximum(_dot(w2_ref[...], h) + b2_ref[...], 0.0)
        h = jnp.maximum(_dot(w3_ref[...], h) + b3_ref[...], 0.0)
        o = _dot(w4_ref[...], h)
        out_ref[:, s] = jax.nn.sigmoid(o + b4_ref[...])


def kernel(x, edge_index, edge_attr, params_rel, params_obj, params_w):
    E, DE = edge_attr.shape
    (W1, b1), (W2, b2), (W3, b3), (W4, b4) = params_w
    H1, H2, H3, DO = W1.shape[0], W2.shape[0], W3.shape[0], W4.shape[0]

    eaT = edge_attr.T

    lanes = 64000
    grid = (pl.cdiv(E, lanes),)

    out = pl.pallas_call(
        _head_kernel,
        grid=grid,
        in_specs=[
            pl.BlockSpec((DE, lanes), lambda i: (0, i)),
            pl.BlockSpec((H1, DE), lambda i: (0, 0)),
            pl.BlockSpec((H1, 1), lambda i: (0, 0)),
            pl.BlockSpec((H2, H1), lambda i: (0, 0)),
            pl.BlockSpec((H2, 1), lambda i: (0, 0)),
            pl.BlockSpec((H3, H2), lambda i: (0, 0)),
            pl.BlockSpec((H3, 1), lambda i: (0, 0)),
            pl.BlockSpec((DO, H3), lambda i: (0, 0)),
            pl.BlockSpec((DO, 1), lambda i: (0, 0)),
        ],
        out_specs=pl.BlockSpec((DO, lanes), lambda i: (0, i)),
        out_shape=jax.ShapeDtypeStruct((DO, E), jnp.float32),
        compiler_params=pltpu.CompilerParams(
            dimension_semantics=("parallel",)),
    )(eaT, W1, b1[:, None], W2, b2[:, None], W3, b3[:, None],
      W4, b4[:, None])
    return out.reshape(E, DO)


# speedup vs baseline: 1.0793x; 1.0004x over previous
import jax
import jax.numpy as jnp
from jax.experimental import pallas as pl
from jax.experimental.pallas import tpu as pltpu


def _dot(a, b):
    return jax.lax.dot_general(
        a, b, (((1,), (0,)), ((), ())),
        precision=jax.lax.Precision.DEFAULT,
        preferred_element_type=jnp.float32)


def _head_kernel(ea_ref, p_ref, out_ref):
    p = p_ref[...]
    w1, b1 = p[:, 0:4], p[:, 4:5]
    w2, b2 = p[:, 5:45], p[:, 45:46]
    w3, b3 = p[:, 46:86], p[:, 86:87]
    w4t, b4 = p[:, 87:88], p[0:1, 88:89]
    h = jnp.maximum(_dot(w1, ea_ref[...]) + b1, 0.0)
    h = jnp.maximum(_dot(w2, h) + b2, 0.0)
    h = jnp.maximum(_dot(w3, h) + b3, 0.0)
    o = jax.lax.dot_general(
        w4t, h, (((0,), (0,)), ((), ())),
        precision=jax.lax.Precision.DEFAULT,
        preferred_element_type=jnp.float32)
    out_ref[...] = jax.nn.sigmoid(o + b4)


def kernel(x, edge_index, edge_attr, params_rel, params_obj, params_w):
    E, DE = edge_attr.shape
    (W1, b1), (W2, b2), (W3, b3), (W4, b4) = params_w
    H = W1.shape[0]
    DO = W4.shape[0]

    eaT = edge_attr.T
    packed = jnp.concatenate(
        [W1, b1[:, None], W2, b2[:, None], W3, b3[:, None], W4.T,
         jnp.full((H, 1), b4[0], jnp.float32)], axis=1)

    lanes = 32000
    grid = (pl.cdiv(E, lanes),)

    out = pl.pallas_call(
        _head_kernel,
        grid=grid,
        in_specs=[
            pl.BlockSpec((DE, lanes), lambda i: (0, i)),
            pl.BlockSpec(packed.shape, lambda i: (0, 0)),
        ],
        out_specs=pl.BlockSpec((DO, lanes), lambda i: (0, i)),
        out_shape=jax.ShapeDtypeStruct((DO, E), jnp.float32),
        compiler_params=pltpu.CompilerParams(
            dimension_semantics=("parallel",)),
    )(eaT, packed)
    return out.reshape(E, DO)
